# baseline (speedup 1.0000x reference)
import jax
import jax.numpy as jnp
from jax.experimental import pallas as pl
from jax.experimental.pallas import tpu as pltpu

N = 10000
E = 160000
IN_DIM = 256
H = 8
C = 64
D = H * C
NEG_SLOPE = 0.2


def _mm_body(x_ref, w_ref, as_ref, ad_ref, xw_ref, asrc_ref, adst_ref):
    xw = jnp.dot(x_ref[...], w_ref[...], preferred_element_type=jnp.float32)
    xw_ref[...] = xw
    asrc_ref[...] = jnp.dot(xw, as_ref[...], preferred_element_type=jnp.float32)
    adst_ref[...] = jnp.dot(xw, ad_ref[...], preferred_element_type=jnp.float32)


def _project(x, W, att_src, att_dst):
    eye = jnp.eye(H, dtype=jnp.float32)
    As = (eye[:, None, :] * att_src[:, :, None]).reshape(D, H)
    Ad = (eye[:, None, :] * att_dst[:, :, None]).reshape(D, H)
    BR = 1000
    grid = (N // BR,)
    return pl.pallas_call(
        _mm_body,
        grid=grid,
        in_specs=[
            pl.BlockSpec((BR, IN_DIM), lambda i: (i, 0)),
            pl.BlockSpec((IN_DIM, D), lambda i: (0, 0)),
            pl.BlockSpec((D, H), lambda i: (0, 0)),
            pl.BlockSpec((D, H), lambda i: (0, 0)),
        ],
        out_specs=[
            pl.BlockSpec((BR, D), lambda i: (i, 0)),
            pl.BlockSpec((BR, H), lambda i: (i, 0)),
            pl.BlockSpec((BR, H), lambda i: (i, 0)),
        ],
        out_shape=[
            jax.ShapeDtypeStruct((N, D), jnp.float32),
            jax.ShapeDtypeStruct((N, H), jnp.float32),
            jax.ShapeDtypeStruct((N, H), jnp.float32),
        ],
    )(x, W, As, Ad)


def kernel(x, edge_index, W, att_src, att_dst, bias):
    xw, a_src, a_dst = _project(x, W, att_src, att_dst)
    src = edge_index[0].astype(jnp.int32)
    dst = edge_index[1].astype(jnp.int32)
    alpha = a_src[src] + a_dst[dst]
    alpha = jax.nn.leaky_relu(alpha, NEG_SLOPE)
    amax = jax.ops.segment_max(alpha, dst, num_segments=N)
    amax = jnp.where(jnp.isfinite(amax), amax, 0.0)
    ex = jnp.exp(alpha - amax[dst])
    denom = jax.ops.segment_sum(ex, dst, num_segments=N)
    coef = ex / (denom[dst] + 1e-16)
    msg = xw.reshape(N, H, C)[src] * coef[..., None]
    out = jax.ops.segment_sum(msg, dst, num_segments=N)
    out = out.reshape(N, D) + bias
    return out
